# baseline (speedup 1.0000x reference)
import jax
import jax.numpy as jnp
from jax.experimental import pallas as pl
from jax.experimental.pallas import tpu as pltpu

TOPK = 64
L_BLK = 1024
C_BLK = 512
ACC_W = 16


def _net(y, d0, desc):
    mx = jnp.maximum if desc else jnp.minimum
    mn = jnp.minimum if desc else jnp.maximum
    m, s, c = y.shape
    d = d0
    while d >= 2:
        h = d // 2
        g = m // (2 * d)
        yr = y.reshape(g, 4, h, s, c)
        t0, t1, t2, t3 = yr[:, 0], yr[:, 1], yr[:, 2], yr[:, 3]
        a, b = mx(t0, t2), mx(t1, t3)
        e, f = mn(t0, t2), mn(t1, t3)
        o0, o1 = mx(a, b), mn(a, b)
        o2, o3 = mx(e, f), mn(e, f)
        y = jnp.concatenate([o0, o1, o2, o3], axis=1).reshape(m, s, c)
        d //= 4
    if d == 1:
        g = m // 2
        yr = y.reshape(g, 2, 1, s, c)
        t, u = yr[:, 0], yr[:, 1]
        y = jnp.concatenate([mx(t, u), mn(t, u)], axis=1).reshape(m, s, c)
    return y


def _grow(qd, qa, desc):
    mx = jnp.maximum if desc else jnp.minimum
    mn = jnp.minimum if desc else jnp.maximum
    m = qd.shape[0]
    if m == 1:
        return jnp.concatenate([mx(qd, qa), mn(qd, qa)], axis=0)
    h = m // 2
    q0, q1, q2, q3 = qd[:h], qd[h:], qa[:h], qa[h:]
    a, b = mx(q0, q2), mx(q1, q3)
    e, f = mn(q0, q2), mn(q1, q3)
    y = jnp.concatenate(
        [mx(a, b), mn(a, b), mx(e, f), mn(e, f)], axis=0
    )
    return _net(y, m // 4, desc) if m >= 4 else y


def _merge_level(xd, xa):
    m, s2, _ = xd.shape
    if 2 * m <= TOPK:
        if s2 == 1:
            return _grow(xd, xa, desc=False)
        return (
            _grow(xd[:, : s2 // 2], xa[:, : s2 // 2], desc=True),
            _grow(xd[:, s2 // 2 :], xa[:, s2 // 2 :], desc=False),
        )
    y = jnp.maximum(xd, xa)
    d0 = TOPK // 2
    if s2 == 1:
        return _net(y, d0, desc=False)
    yd = y[:, : s2 // 2]
    ya = y[:, s2 // 2 :]
    return _net(yd, d0, desc=True), _net(ya, d0, desc=False)


def _topk_kernel(x_ref, o_ref, acc_ref):
    l = pl.program_id(2)
    x = x_ref[0].reshape(1, L_BLK, C_BLK)
    xd, xa = x[:, : L_BLK // 2], x[:, L_BLK // 2 :]
    while xd.shape[0] < TOPK:
        xd, xa = _merge_level(xd, xa)
    hw = ACC_W // 2
    prev_d = jnp.where(l == 0, -jnp.inf, acc_ref[:, :hw])
    prev_a = jnp.where(l == 0, -jnp.inf, acc_ref[:, hw:])
    acc_ref[:, :hw] = _net(jnp.maximum(prev_d, xa), TOPK // 2, desc=True)
    acc_ref[:, hw:] = _net(jnp.maximum(prev_a, xd), TOPK // 2, desc=False)

    @pl.when(l == pl.num_programs(2) - 1)
    def _():
        fd, fa = acc_ref[:, :hw], acc_ref[:, hw:]
        while fd.shape[1] > 1:
            fd, fa = _merge_level(fd, fa)
        y = jnp.maximum(fd, fa)
        o_ref[0] = _net(y, TOPK // 2, desc=True)[:, 0, :]


def kernel(inputs):
    b_dim, l_dim, c_dim = inputs.shape
    grid = (b_dim, c_dim // C_BLK, l_dim // L_BLK)
    return pl.pallas_call(
        _topk_kernel,
        grid=grid,
        in_specs=[pl.BlockSpec((1, L_BLK, C_BLK), lambda b, c, l: (b, l, c))],
        out_specs=pl.BlockSpec((1, TOPK, C_BLK), lambda b, c, l: (b, 0, c)),
        out_shape=jax.ShapeDtypeStruct((b_dim, TOPK, c_dim), jnp.float32),
        scratch_shapes=[pltpu.VMEM((TOPK, ACC_W, C_BLK), jnp.float32)],
        compiler_params=pltpu.CompilerParams(
            dimension_semantics=("parallel", "parallel", "arbitrary"),
        ),
    )(inputs)


# speedup vs baseline: 1.0209x; 1.0209x over previous
import jax
import jax.numpy as jnp
from jax.experimental import pallas as pl
from jax.experimental.pallas import tpu as pltpu

TOPK = 64
L_BLK = 1024
C_BLK = 256
ACC_W = 16


def _net(y, d0, desc):
    mx = jnp.maximum if desc else jnp.minimum
    mn = jnp.minimum if desc else jnp.maximum
    m, s, c = y.shape
    d = d0
    while d >= 2:
        h = d // 2
        g = m // (2 * d)
        yr = y.reshape(g, 4, h, s, c)
        t0, t1, t2, t3 = yr[:, 0], yr[:, 1], yr[:, 2], yr[:, 3]
        a, b = mx(t0, t2), mx(t1, t3)
        e, f = mn(t0, t2), mn(t1, t3)
        o0, o1 = mx(a, b), mn(a, b)
        o2, o3 = mx(e, f), mn(e, f)
        y = jnp.concatenate([o0, o1, o2, o3], axis=1).reshape(m, s, c)
        d //= 4
    if d == 1:
        g = m // 2
        yr = y.reshape(g, 2, 1, s, c)
        t, u = yr[:, 0], yr[:, 1]
        y = jnp.concatenate([mx(t, u), mn(t, u)], axis=1).reshape(m, s, c)
    return y


def _grow(qd, qa, desc):
    mx = jnp.maximum if desc else jnp.minimum
    mn = jnp.minimum if desc else jnp.maximum
    m = qd.shape[0]
    if m == 1:
        return jnp.concatenate([mx(qd, qa), mn(qd, qa)], axis=0)
    h = m // 2
    q0, q1, q2, q3 = qd[:h], qd[h:], qa[:h], qa[h:]
    a, b = mx(q0, q2), mx(q1, q3)
    e, f = mn(q0, q2), mn(q1, q3)
    y = jnp.concatenate(
        [mx(a, b), mn(a, b), mx(e, f), mn(e, f)], axis=0
    )
    return _net(y, m // 4, desc) if m >= 4 else y


def _merge_level(xd, xa):
    m, s2, _ = xd.shape
    if 2 * m <= TOPK:
        if s2 == 1:
            return _grow(xd, xa, desc=False)
        return (
            _grow(xd[:, : s2 // 2], xa[:, : s2 // 2], desc=True),
            _grow(xd[:, s2 // 2 :], xa[:, s2 // 2 :], desc=False),
        )
    y = jnp.maximum(xd, xa)
    d0 = TOPK // 2
    if s2 == 1:
        return _net(y, d0, desc=False)
    yd = y[:, : s2 // 2]
    ya = y[:, s2 // 2 :]
    return _net(yd, d0, desc=True), _net(ya, d0, desc=False)


def _topk_kernel(x_ref, o_ref, acc_ref):
    l = pl.program_id(2)
    x = x_ref[0].reshape(1, L_BLK, C_BLK)
    xd, xa = x[:, : L_BLK // 2], x[:, L_BLK // 2 :]
    while xd.shape[0] < TOPK:
        xd, xa = _merge_level(xd, xa)
    hw = ACC_W // 2
    prev_d = jnp.where(l == 0, -jnp.inf, acc_ref[:, :hw])
    prev_a = jnp.where(l == 0, -jnp.inf, acc_ref[:, hw:])
    acc_ref[:, :hw] = _net(jnp.maximum(prev_d, xa), TOPK // 2, desc=True)
    acc_ref[:, hw:] = _net(jnp.maximum(prev_a, xd), TOPK // 2, desc=False)

    @pl.when(l == pl.num_programs(2) - 1)
    def _():
        fd, fa = acc_ref[:, :hw], acc_ref[:, hw:]
        while fd.shape[1] > 1:
            fd, fa = _merge_level(fd, fa)
        y = jnp.maximum(fd, fa)
        o_ref[0] = _net(y, TOPK // 2, desc=True)[:, 0, :]


def kernel(inputs):
    b_dim, l_dim, c_dim = inputs.shape
    grid = (b_dim, c_dim // C_BLK, l_dim // L_BLK)
    return pl.pallas_call(
        _topk_kernel,
        grid=grid,
        in_specs=[pl.BlockSpec((1, L_BLK, C_BLK), lambda b, c, l: (b, l, c))],
        out_specs=pl.BlockSpec((1, TOPK, C_BLK), lambda b, c, l: (b, 0, c)),
        out_shape=jax.ShapeDtypeStruct((b_dim, TOPK, c_dim), jnp.float32),
        scratch_shapes=[pltpu.VMEM((TOPK, ACC_W, C_BLK), jnp.float32)],
        compiler_params=pltpu.CompilerParams(
            dimension_semantics=("parallel", "parallel", "arbitrary"),
        ),
    )(inputs)
